# Initial kernel scaffold; baseline (speedup 1.0000x reference)
import jax
import jax.numpy as jnp
from jax.experimental import pallas as pl


def kernel(x, edge_index, w1_mu, w1_rho, b1_mu, b1_rho, w2_mu, w2_rho, b2_mu, b2_rho):
    raise NotImplementedError("write your pallas kernel here")


# speedup vs baseline: 8.6640x; 8.6640x over previous
import functools

import jax
import jax.numpy as jnp
from jax import lax
from jax.experimental import pallas as pl
from jax.experimental.pallas import tpu as pltpu
from jax.experimental.pallas import tpu_sc as plsc

N = 10000
D = 128
E = 320000

NC = 2
NS = 16
NW = NC * NS
L = 16

NPAD = 10240
ROWS_PER_TILE = NPAD // NS
E2 = 2 * E
UNIT = 128
CH = 16
NCH = 10
UNITS = CH * NCH
EPT = UNITS * UNIT
EPAD = NW * EPT
PAD_DST = N + 64
BR = 1024

_mesh = plsc.VectorSubcoreMesh(core_axis_name="c", subcore_axis_name="s")


@functools.partial(
    pl.kernel,
    out_type=jax.ShapeDtypeStruct((NW, NPAD), jnp.float32),
    mesh=_mesh,
    scratch_types=[
        pltpu.VMEM((NPAD,), jnp.float32),
        pltpu.VMEM((UNITS, UNIT), jnp.int32),
    ],
    compiler_params=pltpu.CompilerParams(needs_layout_passes=False),
)
def _degree_kernel(dsts_hbm, zvec_hbm, deg_hbm, acc_v, idx_v):
    c = lax.axis_index("c")
    s = lax.axis_index("s")
    wid = c * NS + s
    pltpu.sync_copy(zvec_hbm, acc_v)
    pltpu.sync_copy(dsts_hbm.at[wid], idx_v)
    ones16 = jnp.ones((L,), jnp.float32)

    def body(u, carry):
        for k in range(UNIT // L):
            idx16 = idx_v[u, pl.ds(k * L, L)]
            plsc.addupdate_scatter(acc_v, [idx16], ones16)
        return carry

    lax.fori_loop(0, UNITS, body, 0)
    pltpu.sync_copy(acc_v, deg_hbm.at[wid])


@functools.partial(
    pl.kernel,
    out_type=jax.ShapeDtypeStruct((NC, NPAD, D), jnp.float32),
    mesh=_mesh,
    scratch_types=[
        pltpu.VMEM_SHARED((NPAD, D), jnp.float32),
        pltpu.VMEM((CH, UNIT), jnp.int32),
        pltpu.VMEM((CH, UNIT), jnp.int32),
        pltpu.VMEM((UNIT, D), jnp.float32),
        pltpu.VMEM((UNIT, D), jnp.float32),
        pltpu.SemaphoreType.DMA,
        pltpu.SemaphoreType.DMA,
    ],
)
def _spmm(z_hbm, srcs_hbm, dsts_hbm, zrow_hbm, p_hbm,
          acc, idx_s, idx_d, rows0, rows1, sem0, sem1):
    c = lax.axis_index("c")
    s = lax.axis_index("s")
    wid = c * NS + s

    pltpu.sync_copy(zrow_hbm, acc.at[pl.ds(s * ROWS_PER_TILE, ROWS_PER_TILE)])
    plsc.subcore_barrier()

    def chunk(ci, carry):
        csl = pl.ds(ci * CH, CH)
        pltpu.sync_copy(srcs_hbm.at[wid].at[csl], idx_s)
        pltpu.sync_copy(dsts_hbm.at[wid].at[csl], idx_d)

        def body(j, c2):
            pltpu.async_copy(z_hbm.at[idx_s.at[j]], rows0, sem0)
            pltpu.make_async_copy(z_hbm.at[idx_s.at[j]], rows0, sem0).wait()
            pltpu.sync_copy(rows0, acc.at[idx_d.at[j]], add=True)
            return c2

        return lax.fori_loop(0, CH, body, carry)

    lax.fori_loop(0, NCH, chunk, 0)

    plsc.subcore_barrier()
    sl = pl.ds(s * ROWS_PER_TILE, ROWS_PER_TILE)
    pltpu.sync_copy(acc.at[sl], p_hbm.at[c].at[sl])


def _dinv_block(deg_blk):
    ones = jnp.ones((NW, 1), jnp.float32)
    degsum = lax.dot_general(deg_blk, ones, (((0,), (0,)), ((), ())),
                             preferred_element_type=jnp.float32)
    return lax.rsqrt(degsum + 1.0)


def _tc1_body(x_ref, w_ref, b_ref, deg_ref, z_ref):
    dinv = _dinv_block(deg_ref[...])
    xw = jnp.dot(x_ref[...], w_ref[...], preferred_element_type=jnp.float32)
    z_ref[...] = dinv * (xw + b_ref[...])


def _tc2_body(p_ref, z1_ref, w_ref, b_ref, deg_ref, z2_ref):
    dinv = _dinv_block(deg_ref[...])
    h = jnp.maximum(dinv * (p_ref[0] + p_ref[1] + z1_ref[...]), 0.0)
    hw = jnp.dot(h, w_ref[...], preferred_element_type=jnp.float32)
    z2_ref[...] = dinv * (hw + b_ref[...])


def _tc3_body(p_ref, z2_ref, deg_ref, out_ref):
    dinv = _dinv_block(deg_ref[...])
    out_ref[...] = jnp.maximum(dinv * (p_ref[0] + p_ref[1] + z2_ref[...]), 0.0)


_GRID = (pl.cdiv(N, BR),)
_xspec = pl.BlockSpec((BR, D), lambda i: (i, 0))
_wspec = pl.BlockSpec((D, D), lambda i: (0, 0))
_bspec = pl.BlockSpec((1, D), lambda i: (0, 0))
_degspec = pl.BlockSpec((NW, BR), lambda i: (0, i))
_pspec = pl.BlockSpec((NC, BR, D), lambda i: (0, i, 0))
_o = jax.ShapeDtypeStruct((N, D), jnp.float32)

_tc1 = pl.pallas_call(
    _tc1_body, grid=_GRID, out_shape=_o,
    in_specs=[_xspec, _wspec, _bspec, _degspec], out_specs=_xspec)
_tc2 = pl.pallas_call(
    _tc2_body, grid=_GRID, out_shape=_o,
    in_specs=[_pspec, _xspec, _wspec, _bspec, _degspec], out_specs=_xspec)
_tc3 = pl.pallas_call(
    _tc3_body, grid=_GRID, out_shape=_o,
    in_specs=[_pspec, _xspec, _degspec], out_specs=_xspec)


def _sample(mu, rho, key):
    return mu + jax.nn.softplus(rho) * jax.random.normal(key, mu.shape, mu.dtype)


def kernel(x, edge_index, w1_mu, w1_rho, b1_mu, b1_rho,
           w2_mu, w2_rho, b2_mu, b2_rho):
    ei = edge_index.astype(jnp.int32)
    row0, row1 = ei[0], ei[1]
    npad_e = EPAD - E2
    srcs = jnp.concatenate(
        [row1, row0, jnp.zeros((npad_e,), jnp.int32)]).reshape(NW, UNITS, UNIT)
    dsts = jnp.concatenate(
        [row0, row1,
         jnp.full((npad_e,), PAD_DST, jnp.int32)]).reshape(NW, UNITS, UNIT)

    k1, k2 = jax.random.key(1), jax.random.key(2)
    w1t = _sample(w1_mu, w1_rho, k1).T
    b1 = _sample(b1_mu, b1_rho, jax.random.fold_in(k1, 1)).reshape(1, D)
    w2t = _sample(w2_mu, w2_rho, k2).T
    b2 = _sample(b2_mu, b2_rho, jax.random.fold_in(k2, 1)).reshape(1, D)

    zvec = jnp.zeros((NPAD,), jnp.float32)
    zrow = jnp.zeros((ROWS_PER_TILE, D), jnp.float32)

    deg_parts = _degree_kernel(dsts, zvec)
    z1 = _tc1(x, w1t, b1, deg_parts)
    p1 = _spmm(z1, srcs, dsts, zrow)
    z2 = _tc2(p1, z1, w2t, b2, deg_parts)
    p2 = _spmm(z2, srcs, dsts, zrow)
    return _tc3(p2, z2, deg_parts)


# speedup vs baseline: 9.4480x; 1.0905x over previous
import functools

import jax
import jax.numpy as jnp
from jax import lax
from jax.experimental import pallas as pl
from jax.experimental.pallas import tpu as pltpu
from jax.experimental.pallas import tpu_sc as plsc

N = 10000
D = 128
E = 320000

NC = 2
NS = 16
NW = NC * NS
L = 16

NPAD = 10240
ROWS_PER_TILE = NPAD // NS
E2 = 2 * E
UNIT = 128
CH = 16
NCH = 10
UNITS = CH * NCH
EPT = UNITS * UNIT
EPAD = NW * EPT
PAD_DST = N + 64
BR = 1024

_mesh = plsc.VectorSubcoreMesh(core_axis_name="c", subcore_axis_name="s")


@functools.partial(
    pl.kernel,
    out_type=jax.ShapeDtypeStruct((NW, NPAD), jnp.float32),
    mesh=_mesh,
    scratch_types=[
        pltpu.VMEM((NPAD,), jnp.float32),
        pltpu.VMEM((UNITS, UNIT), jnp.int32),
    ],
    compiler_params=pltpu.CompilerParams(needs_layout_passes=False),
)
def _degree_kernel(dsts_hbm, zvec_hbm, deg_hbm, acc_v, idx_v):
    c = lax.axis_index("c")
    s = lax.axis_index("s")
    wid = c * NS + s
    pltpu.sync_copy(zvec_hbm, acc_v)
    pltpu.sync_copy(dsts_hbm.at[wid], idx_v)
    ones16 = jnp.ones((L,), jnp.float32)

    def body(u, carry):
        for k in range(UNIT // L):
            idx16 = idx_v[u, pl.ds(k * L, L)]
            plsc.addupdate_scatter(acc_v, [idx16], ones16)
        return carry

    lax.fori_loop(0, UNITS, body, 0)
    pltpu.sync_copy(acc_v, deg_hbm.at[wid])


@functools.partial(
    pl.kernel,
    out_type=jax.ShapeDtypeStruct((NC, NPAD, D), jnp.float32),
    mesh=_mesh,
    scratch_types=[
        pltpu.VMEM_SHARED((NPAD, D), jnp.float32),
        pltpu.VMEM((CH, UNIT), jnp.int32),
        pltpu.VMEM((CH, UNIT), jnp.int32),
        pltpu.VMEM((UNIT, D), jnp.float32),
        pltpu.VMEM((UNIT, D), jnp.float32),
        pltpu.SemaphoreType.DMA,
        pltpu.SemaphoreType.DMA,
    ],
)
def _spmm(z_hbm, srcs_hbm, dsts_hbm, zrow_hbm, p_hbm,
          acc, idx_s, idx_d, rows0, rows1, sem0, sem1):
    c = lax.axis_index("c")
    s = lax.axis_index("s")
    wid = c * NS + s

    pltpu.sync_copy(zrow_hbm, acc.at[pl.ds(s * ROWS_PER_TILE, ROWS_PER_TILE)])
    plsc.subcore_barrier()

    rows = (rows0, rows1)
    sems = (sem0, sem1)

    def chunk(ci, carry):
        csl = pl.ds(ci * CH, CH)
        pltpu.sync_copy(srcs_hbm.at[wid].at[csl], idx_s)
        pltpu.sync_copy(dsts_hbm.at[wid].at[csl], idx_d)
        pltpu.async_copy(z_hbm.at[idx_s.at[0]], rows[0], sems[0])
        for u in range(CH):
            b = u % 2
            if u + 1 < CH:
                pltpu.async_copy(z_hbm.at[idx_s.at[u + 1]], rows[1 - b],
                                 sems[1 - b])
            pltpu.make_async_copy(z_hbm.at[idx_s.at[u]], rows[b],
                                  sems[b]).wait()
            pltpu.sync_copy(rows[b], acc.at[idx_d.at[u]], add=True)
        return carry

    lax.fori_loop(0, NCH, chunk, 0)

    plsc.subcore_barrier()
    sl = pl.ds(s * ROWS_PER_TILE, ROWS_PER_TILE)
    pltpu.sync_copy(acc.at[sl], p_hbm.at[c].at[sl])


def _dinv_block(deg_blk):
    ones = jnp.ones((NW, 1), jnp.float32)
    degsum = lax.dot_general(deg_blk, ones, (((0,), (0,)), ((), ())),
                             preferred_element_type=jnp.float32)
    return lax.rsqrt(degsum + 1.0)


def _tc1_body(x_ref, w_ref, b_ref, deg_ref, z_ref):
    dinv = _dinv_block(deg_ref[...])
    xw = jnp.dot(x_ref[...], w_ref[...], preferred_element_type=jnp.float32)
    z_ref[...] = dinv * (xw + b_ref[...])


def _tc2_body(p_ref, z1_ref, w_ref, b_ref, deg_ref, z2_ref):
    dinv = _dinv_block(deg_ref[...])
    h = jnp.maximum(dinv * (p_ref[0] + p_ref[1] + z1_ref[...]), 0.0)
    hw = jnp.dot(h, w_ref[...], preferred_element_type=jnp.float32)
    z2_ref[...] = dinv * (hw + b_ref[...])


def _tc3_body(p_ref, z2_ref, deg_ref, out_ref):
    dinv = _dinv_block(deg_ref[...])
    out_ref[...] = jnp.maximum(dinv * (p_ref[0] + p_ref[1] + z2_ref[...]), 0.0)


_GRID = (pl.cdiv(N, BR),)
_xspec = pl.BlockSpec((BR, D), lambda i: (i, 0))
_wspec = pl.BlockSpec((D, D), lambda i: (0, 0))
_bspec = pl.BlockSpec((1, D), lambda i: (0, 0))
_degspec = pl.BlockSpec((NW, BR), lambda i: (0, i))
_pspec = pl.BlockSpec((NC, BR, D), lambda i: (0, i, 0))
_o = jax.ShapeDtypeStruct((N, D), jnp.float32)

_tc1 = pl.pallas_call(
    _tc1_body, grid=_GRID, out_shape=_o,
    in_specs=[_xspec, _wspec, _bspec, _degspec], out_specs=_xspec)
_tc2 = pl.pallas_call(
    _tc2_body, grid=_GRID, out_shape=_o,
    in_specs=[_pspec, _xspec, _wspec, _bspec, _degspec], out_specs=_xspec)
_tc3 = pl.pallas_call(
    _tc3_body, grid=_GRID, out_shape=_o,
    in_specs=[_pspec, _xspec, _degspec], out_specs=_xspec)


def _sample(mu, rho, key):
    return mu + jax.nn.softplus(rho) * jax.random.normal(key, mu.shape, mu.dtype)


def kernel(x, edge_index, w1_mu, w1_rho, b1_mu, b1_rho,
           w2_mu, w2_rho, b2_mu, b2_rho):
    ei = edge_index.astype(jnp.int32)
    row0, row1 = ei[0], ei[1]
    npad_e = EPAD - E2
    srcs = jnp.concatenate(
        [row1, row0, jnp.zeros((npad_e,), jnp.int32)]).reshape(NW, UNITS, UNIT)
    dsts = jnp.concatenate(
        [row0, row1,
         jnp.full((npad_e,), PAD_DST, jnp.int32)]).reshape(NW, UNITS, UNIT)

    k1, k2 = jax.random.key(1), jax.random.key(2)
    w1t = _sample(w1_mu, w1_rho, k1).T
    b1 = _sample(b1_mu, b1_rho, jax.random.fold_in(k1, 1)).reshape(1, D)
    w2t = _sample(w2_mu, w2_rho, k2).T
    b2 = _sample(b2_mu, b2_rho, jax.random.fold_in(k2, 1)).reshape(1, D)

    zvec = jnp.zeros((NPAD,), jnp.float32)
    zrow = jnp.zeros((ROWS_PER_TILE, D), jnp.float32)

    deg_parts = _degree_kernel(dsts, zvec)
    z1 = _tc1(x, w1t, b1, deg_parts)
    p1 = _spmm(z1, srcs, dsts, zrow)
    z2 = _tc2(p1, z1, w2t, b2, deg_parts)
    p2 = _spmm(z2, srcs, dsts, zrow)
    return _tc3(p2, z2, deg_parts)


# speedup vs baseline: 9.5833x; 1.0143x over previous
import functools

import jax
import jax.numpy as jnp
from jax import lax
from jax.experimental import pallas as pl
from jax.experimental.pallas import tpu as pltpu
from jax.experimental.pallas import tpu_sc as plsc

N = 10000
D = 128
E = 320000

NC = 2
NS = 16
NW = NC * NS
L = 16

NPAD = 10240
ROWS_PER_TILE = NPAD // NS
E2 = 2 * E
UNIT = 64
CH = 16
NCH = 20
UNITS = CH * NCH
EPT = UNITS * UNIT
EPAD = NW * EPT
PAD_DST = N + 64
BR = 1024

_mesh = plsc.VectorSubcoreMesh(core_axis_name="c", subcore_axis_name="s")


@functools.partial(
    pl.kernel,
    out_type=jax.ShapeDtypeStruct((NW, NPAD), jnp.float32),
    mesh=_mesh,
    scratch_types=[
        pltpu.VMEM((NPAD,), jnp.float32),
        pltpu.VMEM((UNITS, UNIT), jnp.int32),
    ],
    compiler_params=pltpu.CompilerParams(needs_layout_passes=False),
)
def _degree_kernel(dsts_hbm, zvec_hbm, deg_hbm, acc_v, idx_v):
    c = lax.axis_index("c")
    s = lax.axis_index("s")
    wid = c * NS + s
    pltpu.sync_copy(zvec_hbm, acc_v)
    pltpu.sync_copy(dsts_hbm.at[wid], idx_v)
    ones16 = jnp.ones((L,), jnp.float32)

    def body(u, carry):
        for k in range(UNIT // L):
            idx16 = idx_v[u, pl.ds(k * L, L)]
            plsc.addupdate_scatter(acc_v, [idx16], ones16)
        return carry

    lax.fori_loop(0, UNITS, body, 0)
    pltpu.sync_copy(acc_v, deg_hbm.at[wid])


@functools.partial(
    pl.kernel,
    out_type=jax.ShapeDtypeStruct((NC, NPAD, D), jnp.float32),
    mesh=_mesh,
    scratch_types=[
        pltpu.VMEM_SHARED((NPAD, D), jnp.float32),
        pltpu.VMEM((CH, UNIT), jnp.int32),
        pltpu.VMEM((CH, UNIT), jnp.int32),
        pltpu.VMEM((UNIT, D), jnp.float32),
        pltpu.VMEM((UNIT, D), jnp.float32),
        pltpu.VMEM((UNIT, D), jnp.float32),
        pltpu.VMEM((UNIT, D), jnp.float32),
        pltpu.SemaphoreType.DMA,
        pltpu.SemaphoreType.DMA,
        pltpu.SemaphoreType.DMA,
        pltpu.SemaphoreType.DMA,
    ],
)
def _spmm(z_hbm, srcs_hbm, dsts_hbm, zrow_hbm, p_hbm,
          acc, idx_s, idx_d, rows0, rows1, rows2, rows3,
          sem0, sem1, sem2, sem3):
    c = lax.axis_index("c")
    s = lax.axis_index("s")
    wid = c * NS + s
    rows = (rows0, rows1, rows2, rows3)
    sems = (sem0, sem1, sem2, sem3)

    pltpu.sync_copy(zrow_hbm, acc.at[pl.ds(s * ROWS_PER_TILE, ROWS_PER_TILE)])
    plsc.subcore_barrier()

    def _gather(u, b):
        return pltpu.async_copy(z_hbm.at[idx_s.at[u]], rows[b], sems[b])

    def _scatter(u, b):
        return pltpu.async_copy(rows[b], acc.at[idx_d.at[u]], sems[b],
                                add=True)

    def chunk(ci, carry):
        csl = pl.ds(ci * CH, CH)
        pltpu.sync_copy(srcs_hbm.at[wid].at[csl], idx_s)
        pltpu.sync_copy(dsts_hbm.at[wid].at[csl], idx_d)
        _gather(0, 0)
        _gather(1, 1)
        for u in range(CH):
            b = u % 4
            pltpu.make_async_copy(z_hbm.at[idx_s.at[u]], rows[b],
                                  sems[b]).wait()
            _scatter(u, b)
            if u + 2 < CH:
                nb = (u + 2) % 4
                if u >= 2:
                    pltpu.make_async_copy(
                        rows[nb], acc.at[idx_d.at[u - 2]],
                        sems[nb]).wait()
                _gather(u + 2, nb)
        for u in range(CH - 4, CH):
            b = u % 4
            pltpu.make_async_copy(rows[b], acc.at[idx_d.at[u]],
                                  sems[b]).wait()
        return carry

    lax.fori_loop(0, NCH, chunk, 0)

    plsc.subcore_barrier()
    sl = pl.ds(s * ROWS_PER_TILE, ROWS_PER_TILE)
    pltpu.sync_copy(acc.at[sl], p_hbm.at[c].at[sl])


def _dinv_block(deg_blk):
    ones = jnp.ones((NW, 1), jnp.float32)
    degsum = lax.dot_general(deg_blk, ones, (((0,), (0,)), ((), ())),
                             preferred_element_type=jnp.float32)
    return lax.rsqrt(degsum + 1.0)


def _tc1_body(x_ref, w_ref, b_ref, deg_ref, z_ref):
    dinv = _dinv_block(deg_ref[...])
    xw = jnp.dot(x_ref[...], w_ref[...], preferred_element_type=jnp.float32)
    z_ref[...] = dinv * (xw + b_ref[...])


def _tc2_body(p_ref, z1_ref, w_ref, b_ref, deg_ref, z2_ref):
    dinv = _dinv_block(deg_ref[...])
    h = jnp.maximum(dinv * (p_ref[0] + p_ref[1] + z1_ref[...]), 0.0)
    hw = jnp.dot(h, w_ref[...], preferred_element_type=jnp.float32)
    z2_ref[...] = dinv * (hw + b_ref[...])


def _tc3_body(p_ref, z2_ref, deg_ref, out_ref):
    dinv = _dinv_block(deg_ref[...])
    out_ref[...] = jnp.maximum(dinv * (p_ref[0] + p_ref[1] + z2_ref[...]), 0.0)


_GRID = (pl.cdiv(N, BR),)
_xspec = pl.BlockSpec((BR, D), lambda i: (i, 0))
_wspec = pl.BlockSpec((D, D), lambda i: (0, 0))
_bspec = pl.BlockSpec((1, D), lambda i: (0, 0))
_degspec = pl.BlockSpec((NW, BR), lambda i: (0, i))
_pspec = pl.BlockSpec((NC, BR, D), lambda i: (0, i, 0))
_o = jax.ShapeDtypeStruct((N, D), jnp.float32)

_tc1 = pl.pallas_call(
    _tc1_body, grid=_GRID, out_shape=_o,
    in_specs=[_xspec, _wspec, _bspec, _degspec], out_specs=_xspec)
_tc2 = pl.pallas_call(
    _tc2_body, grid=_GRID, out_shape=_o,
    in_specs=[_pspec, _xspec, _wspec, _bspec, _degspec], out_specs=_xspec)
_tc3 = pl.pallas_call(
    _tc3_body, grid=_GRID, out_shape=_o,
    in_specs=[_pspec, _xspec, _degspec], out_specs=_xspec)


def _sample(mu, rho, key):
    return mu + jax.nn.softplus(rho) * jax.random.normal(key, mu.shape, mu.dtype)


def kernel(x, edge_index, w1_mu, w1_rho, b1_mu, b1_rho,
           w2_mu, w2_rho, b2_mu, b2_rho):
    ei = edge_index.astype(jnp.int32)
    row0, row1 = ei[0], ei[1]
    npad_e = EPAD - E2
    srcs = jnp.concatenate(
        [row1, row0, jnp.zeros((npad_e,), jnp.int32)]).reshape(NW, UNITS, UNIT)
    dsts = jnp.concatenate(
        [row0, row1,
         jnp.full((npad_e,), PAD_DST, jnp.int32)]).reshape(NW, UNITS, UNIT)

    k1, k2 = jax.random.key(1), jax.random.key(2)
    w1t = _sample(w1_mu, w1_rho, k1).T
    b1 = _sample(b1_mu, b1_rho, jax.random.fold_in(k1, 1)).reshape(1, D)
    w2t = _sample(w2_mu, w2_rho, k2).T
    b2 = _sample(b2_mu, b2_rho, jax.random.fold_in(k2, 1)).reshape(1, D)

    zvec = jnp.zeros((NPAD,), jnp.float32)
    zrow = jnp.zeros((ROWS_PER_TILE, D), jnp.float32)

    deg_parts = _degree_kernel(dsts, zvec)
    z1 = _tc1(x, w1t, b1, deg_parts)
    p1 = _spmm(z1, srcs, dsts, zrow)
    z2 = _tc2(p1, z1, w2t, b2, deg_parts)
    p2 = _spmm(z2, srcs, dsts, zrow)
    return _tc3(p2, z2, deg_parts)
